# baseline (speedup 1.0000x reference)
import jax
import jax.numpy as jnp
from jax import lax
from jax.experimental import pallas as pl
from jax.experimental.pallas import tpu as pltpu
from jax.experimental.pallas import tpu_sc as plsc

NUM_EMB = 1000000
DIM = 32
BATCH = 4096
HIST = 50

NC = 2
NS = 16
NW = NC * NS
BAGS_PER_TILE = BATCH // NW
BAGS_PER_CHUNK = 2
CHUNK_ROWS = BAGS_PER_CHUNK * HIST
CHUNK_PAD = 128
CHUNKS = BAGS_PER_TILE // BAGS_PER_CHUNK
IDX_ROWS = BATCH * HIST // CHUNK_ROWS
ROWS_PER_TILE = IDX_ROWS // NW
PACKED_ROWS = NUM_EMB // 4
OUT_ROWS = BATCH // 4

NBUF = 4


def _splat(v):
    return jnp.full((16,), v, dtype=jnp.int32)


def _bag_mean(buf, colb_v, c, base):
    iota = lax.iota(jnp.int32, 16)
    cvec = _splat(c)

    def seg(r, h):
        colb = plsc.load_gather(colb_v, [cvec, _splat(base + r)])
        col = colb + (iota + (16 * h))
        return plsc.load_gather(buf, [_splat(base + r), col])

    h0 = [seg(k, 0) for k in range(4)]
    h1 = [seg(k, 1) for k in range(4)]
    for l in range(4, HIST):
        h0[l & 3] = h0[l & 3] + seg(l, 0)
        h1[l & 3] = h1[l & 3] + seg(l, 1)
    s0 = (h0[0] + h0[1]) + (h0[2] + h0[3])
    s1 = (h1[0] + h1[1]) + (h1[2] + h1[3])
    inv = jnp.float32(1.0 / HIST)
    return s0 * inv, s1 * inv


def _sc_body(idxp_hbm, colb_hbm, table_hbm, out_hbm, idxp_v, colb_v, bufs,
             out_v, sems):
    wid = lax.axis_index("s") * NC + lax.axis_index("c")
    row0 = wid * ROWS_PER_TILE

    pltpu.sync_copy(idxp_hbm.at[pl.ds(row0, ROWS_PER_TILE), :], idxp_v)
    pltpu.sync_copy(colb_hbm.at[pl.ds(row0, ROWS_PER_TILE), :], colb_v)

    def _start(c, slot):
        pltpu.make_async_copy(
            table_hbm.at[idxp_v.at[c]], bufs[slot], sems[slot]).start()

    def _wait(c, slot):
        pltpu.make_async_copy(
            table_hbm.at[idxp_v.at[c]], bufs[slot], sems[slot]).wait()

    for s in range(NBUF):
        _start(s, s)

    iota = lax.iota(jnp.int32, 16)

    @pl.loop(0, CHUNKS, step=NBUF)
    def _chunk(c):
        for b in range(NBUF):
            cc = c + b
            _wait(cc, b)
            buf = bufs[b]
            for i in range(BAGS_PER_CHUNK):
                s0, s1 = _bag_mean(buf, colb_v, cc, i * HIST)
                r = cc * BAGS_PER_CHUNK + i
                prow = _splat(r >> 2)
                pcol = _splat((r & 3) * DIM) + iota
                plsc.store_scatter(out_v, [prow, pcol], s0)
                plsc.store_scatter(out_v, [prow, pcol + 16], s1)

            @pl.when(cc + NBUF < CHUNKS)
            def _():
                _start(cc + NBUF, b)

    pltpu.sync_copy(out_v, out_hbm.at[pl.ds(wid * (BAGS_PER_TILE // 4),
                                            BAGS_PER_TILE // 4), :])


@jax.jit
def _sc_call(idxp, colb, table4):
    mesh = plsc.VectorSubcoreMesh(core_axis_name="c", subcore_axis_name="s")
    return pl.kernel(
        _sc_body,
        out_type=jax.ShapeDtypeStruct((OUT_ROWS, 4 * DIM), jnp.float32),
        mesh=mesh,
        compiler_params=pltpu.CompilerParams(use_tc_tiling_on_sc=True,
                                             needs_layout_passes=False),
        scratch_types=[
            pltpu.VMEM((ROWS_PER_TILE, CHUNK_PAD), jnp.int32),
            pltpu.VMEM((ROWS_PER_TILE, CHUNK_PAD), jnp.int32),
            [pltpu.VMEM((CHUNK_PAD, 4 * DIM), jnp.float32)] * NBUF,
            pltpu.VMEM((BAGS_PER_TILE // 4, 4 * DIM), jnp.float32),
            [pltpu.SemaphoreType.DMA] * NBUF,
        ],
    )(idxp, colb, table4)


def kernel(input, weight):
    idx = input.astype(jnp.int32).reshape(IDX_ROWS, CHUNK_ROWS)
    idx = jnp.pad(idx, ((0, 0), (0, CHUNK_PAD - CHUNK_ROWS)))
    idxp = idx >> 2
    colb = (idx & 3) * DIM
    table4 = weight.reshape(PACKED_ROWS, 4 * DIM)
    out = _sc_call(idxp, colb, table4)
    return out.reshape(BATCH, DIM)


# speedup vs baseline: 4.5443x; 4.5443x over previous
import functools

import jax
import jax.numpy as jnp
from jax import lax
from jax.experimental import pallas as pl
from jax.experimental.pallas import tpu as pltpu
from jax.experimental.pallas import tpu_sc as plsc

NUM_EMB = 1000000
DIM = 32
BATCH = 4096
HIST = 50

NC = 2
NS = 16
NW = NC * NS
BAGS_PER_TILE = BATCH // NW
BAGS_PER_CHUNK = 2
CHUNK_ROWS = BAGS_PER_CHUNK * HIST
CHUNK_PAD = 104
CHUNKS = BAGS_PER_TILE // BAGS_PER_CHUNK
IDX_ROWS = BATCH * HIST // CHUNK_ROWS
ROWS_PER_TILE = IDX_ROWS // NW


def _bag_mean(buf, base):
    h0 = [buf[base + k, 0:16] for k in range(4)]
    h1 = [buf[base + k, 16:32] for k in range(4)]
    for l in range(4, HIST):
        h0[l & 3] = h0[l & 3] + buf[base + l, 0:16]
        h1[l & 3] = h1[l & 3] + buf[base + l, 16:32]
    s0 = (h0[0] + h0[1]) + (h0[2] + h0[3])
    s1 = (h1[0] + h1[1]) + (h1[2] + h1[3])
    inv = jnp.float32(1.0 / HIST)
    return s0 * inv, s1 * inv


NBUF = 8


def _sc_body(idx_hbm, table_hbm, out_hbm, idx_v, bufs, out_v, sems):
    wid = lax.axis_index("s") * NC + lax.axis_index("c")
    row0 = wid * ROWS_PER_TILE

    pltpu.sync_copy(idx_hbm.at[pl.ds(row0, ROWS_PER_TILE), :], idx_v)

    def _start(c, slot):
        pltpu.make_async_copy(
            table_hbm.at[idx_v.at[c]], bufs[slot], sems[slot]).start()

    def _wait(c, slot):
        pltpu.make_async_copy(
            table_hbm.at[idx_v.at[c]], bufs[slot], sems[slot]).wait()

    for s in range(NBUF):
        _start(s, s)

    @pl.loop(0, CHUNKS, step=NBUF)
    def _chunk(c):
        for b in range(NBUF):
            cc = c + b
            _wait(cc, b)
            buf = bufs[b]
            for i in range(BAGS_PER_CHUNK):
                s0, s1 = _bag_mean(buf, i * HIST)
                r = cc * BAGS_PER_CHUNK + i
                out_v[r, 0:16] = s0
                out_v[r, 16:32] = s1

            @pl.when(cc + NBUF < CHUNKS)
            def _():
                _start(cc + NBUF, b)

    pltpu.sync_copy(out_v, out_hbm.at[pl.ds(wid * BAGS_PER_TILE,
                                            BAGS_PER_TILE), :])


@jax.jit
def _sc_call(idx_p, weight):
    mesh = plsc.VectorSubcoreMesh(core_axis_name="c", subcore_axis_name="s")
    return pl.kernel(
        _sc_body,
        out_type=jax.ShapeDtypeStruct((BATCH, DIM), jnp.float32),
        mesh=mesh,
        compiler_params=pltpu.CompilerParams(use_tc_tiling_on_sc=False),
        scratch_types=[
            pltpu.VMEM((ROWS_PER_TILE, CHUNK_PAD), jnp.int32),
            [pltpu.VMEM((CHUNK_PAD, DIM), jnp.float32)] * NBUF,
            pltpu.VMEM((BAGS_PER_TILE, DIM), jnp.float32),
            [pltpu.SemaphoreType.DMA] * NBUF,
        ],
    )(idx_p, weight)


def kernel(input, weight):
    idx = input.astype(jnp.int32).reshape(IDX_ROWS, CHUNK_ROWS)
    idx_p = jnp.pad(idx, ((0, 0), (0, CHUNK_PAD - CHUNK_ROWS)))
    return _sc_call(idx_p, weight)
